# baseline (speedup 1.0000x reference)
import functools

import jax
import jax.numpy as jnp
from jax import lax
from jax.experimental import pallas as pl
from jax.experimental.pallas import tpu as pltpu
from jax.experimental.pallas import tpu_sc as plsc

NC = 2
NS = 16
NW = NC * NS
CHUNK = 128
D = 128


@functools.partial(jax.jit, static_argnums=(2,))
def _emb_sc(m_flat, table, b):
    chunks_per_w = b // (NW * CHUNK)
    mesh = plsc.VectorSubcoreMesh(
        core_axis_name="c", subcore_axis_name="s",
        num_cores=NC, num_subcores=NS)

    @functools.partial(
        pl.kernel,
        out_type=jax.ShapeDtypeStruct((b, D), jnp.float32),
        mesh=mesh,
        scratch_types=[
            pltpu.VMEM((CHUNK,), jnp.int32),
            pltpu.VMEM((CHUNK, D), jnp.float32),
            pltpu.SemaphoreType.DMA,
        ],
    )
    def k(m_hbm, table_hbm, out_hbm, idx_v, rows_v, sem):
        wid = lax.axis_index("s") * NC + lax.axis_index("c")
        base0 = wid * chunks_per_w * CHUNK

        def body(j, carry):
            base = base0 + j * CHUNK
            pltpu.sync_copy(m_hbm.at[pl.ds(base, CHUNK)], idx_v)
            pltpu.async_copy(table_hbm.at[idx_v], rows_v, sem).wait()
            pltpu.sync_copy(rows_v, out_hbm.at[pl.ds(base, CHUNK)])
            return carry

        lax.fori_loop(0, chunks_per_w, body, 0)

    return k(m_flat, table)


def kernel(m, table):
    b = m.shape[0] * m.shape[1]
    m_flat = m.reshape(b).astype(jnp.int32)
    out = _emb_sc(m_flat, table, b)
    return out.reshape(m.shape[0], m.shape[1], D)


# speedup vs baseline: 1.0328x; 1.0328x over previous
import functools

import jax
import jax.numpy as jnp
from jax import lax
from jax.experimental import pallas as pl
from jax.experimental.pallas import tpu as pltpu
from jax.experimental.pallas import tpu_sc as plsc

NC = 2
NS = 16
NW = NC * NS
D = 128
CHUNK = 256
SLAB_CHUNKS = 8
SLAB = CHUNK * SLAB_CHUNKS


@functools.partial(jax.jit, static_argnums=(2,))
def _emb_sc(m_flat, table, b):
    rows_per_w = b // NW
    slabs = rows_per_w // SLAB
    mesh = plsc.VectorSubcoreMesh(
        core_axis_name="c", subcore_axis_name="s",
        num_cores=NC, num_subcores=NS)

    @functools.partial(
        pl.kernel,
        out_type=jax.ShapeDtypeStruct((b, D), jnp.float32),
        mesh=mesh,
        compiler_params=pltpu.CompilerParams(needs_layout_passes=False),
        scratch_types=[
            pltpu.VMEM((10, D), jnp.float32),
            pltpu.VMEM((SLAB,), jnp.int32),
            pltpu.VMEM((CHUNK, D), jnp.float32),
            pltpu.VMEM((CHUNK, D), jnp.float32),
            pltpu.SemaphoreType.DMA,
            pltpu.SemaphoreType.DMA,
        ],
    )
    def k(m_hbm, table_hbm, out_hbm, tab_v, idx_v, ob0, ob1, ws0, ws1):
        wid = lax.axis_index("s") * NC + lax.axis_index("c")
        row0 = wid * rows_per_w
        pltpu.sync_copy(table_hbm, tab_v)
        lane = lax.iota(jnp.int32, 16)
        bufs = (ob0, ob1)
        sems = (ws0, ws1)

        def expand_chunk(cc, ob):
            def g_body(g, _):
                idx16 = idx_v[pl.ds(cc * CHUNK + g * 16, 16)]
                row16 = g * 16 + lane
                for c in range(D):
                    cvec = jnp.full((16,), c, jnp.int32)
                    vals = plsc.load_gather(tab_v, [idx16, cvec])
                    plsc.store_scatter(ob, [row16, cvec], vals)
                return 0
            lax.fori_loop(0, CHUNK // 16, g_body, 0)

        def slab_body(s, _):
            pltpu.sync_copy(m_hbm.at[pl.ds(row0 + s * SLAB, SLAB)], idx_v)

            def pair_body(p, _):
                for bsel in range(2):
                    cc = p * 2 + bsel
                    ob, ws = bufs[bsel], sems[bsel]
                    rbase = row0 + s * SLAB + cc * CHUNK
                    dst = out_hbm.at[pl.ds(rbase, CHUNK)]

                    @pl.when(s + p > 0)
                    def _():
                        pltpu.make_async_copy(ob, dst, ws).wait()

                    expand_chunk(cc, ob)
                    pltpu.async_copy(ob, dst, ws)
                return 0

            lax.fori_loop(0, SLAB_CHUNKS // 2, pair_body, 0)
            return 0

        lax.fori_loop(0, slabs, slab_body, 0)
        tail = row0 + rows_per_w - 2 * CHUNK
        for bsel in range(2):
            dst = out_hbm.at[pl.ds(tail + bsel * CHUNK, CHUNK)]
            pltpu.make_async_copy(bufs[bsel], dst, sems[bsel]).wait()

    return k(m_flat, table)


def kernel(m, table):
    b = m.shape[0] * m.shape[1]
    m_flat = m.reshape(b).astype(jnp.int32)
    out = _emb_sc(m_flat, table, b)
    return out.reshape(m.shape[0], m.shape[1], D)


# speedup vs baseline: 1.6543x; 1.6017x over previous
import functools

import jax
import jax.numpy as jnp
from jax import lax
from jax.experimental import pallas as pl
from jax.experimental.pallas import tpu as pltpu
from jax.experimental.pallas import tpu_sc as plsc

NC = 2
NS = 16
NW = NC * NS
D = 128
CHUNK = 256
SLAB_CHUNKS = 8
SLAB = CHUNK * SLAB_CHUNKS


@functools.partial(jax.jit, static_argnums=(2,))
def _emb_sc(m_flat, table, b):
    rows_per_w = b // NW
    slabs = rows_per_w // SLAB
    mesh = plsc.VectorSubcoreMesh(
        core_axis_name="c", subcore_axis_name="s",
        num_cores=NC, num_subcores=NS)

    @functools.partial(
        pl.kernel,
        out_type=jax.ShapeDtypeStruct((b, D), jnp.float32),
        mesh=mesh,
        compiler_params=pltpu.CompilerParams(needs_layout_passes=False),
        scratch_types=[
            pltpu.VMEM((10, D), jnp.float32),
            pltpu.VMEM((SLAB,), jnp.int32),
            pltpu.VMEM((CHUNK, D), jnp.float32),
            pltpu.VMEM((CHUNK, D), jnp.float32),
            pltpu.SemaphoreType.DMA,
            pltpu.SemaphoreType.DMA,
        ],
    )
    def k(m_hbm, table_hbm, out_hbm, tab_v, idx_v, ob0, ob1, ws0, ws1):
        wid = lax.axis_index("s") * NC + lax.axis_index("c")
        row0 = wid * rows_per_w
        pltpu.sync_copy(table_hbm, tab_v)
        lane = lax.iota(jnp.int32, 16)
        bufs = (ob0, ob1)
        sems = (ws0, ws1)

        def expand_chunk(cc, ob):
            @plsc.parallel_loop(0, CHUNK // 16)
            def g_body(g):
                idx16 = idx_v[pl.ds(cc * CHUNK + g * 16, 16)]
                row16 = g * 16 + lane

                @plsc.parallel_loop(0, D, unroll=8)
                def c_body(c):
                    cvec = jnp.zeros((16,), jnp.int32) + c
                    vals = plsc.load_gather(tab_v, [idx16, cvec])
                    plsc.store_scatter(ob, [row16, cvec], vals)

        def slab_body(s, _):
            pltpu.sync_copy(m_hbm.at[pl.ds(row0 + s * SLAB, SLAB)], idx_v)

            def pair_body(p, _):
                for bsel in range(2):
                    cc = p * 2 + bsel
                    ob, ws = bufs[bsel], sems[bsel]
                    rbase = row0 + s * SLAB + cc * CHUNK
                    dst = out_hbm.at[pl.ds(rbase, CHUNK)]

                    @pl.when(s + p > 0)
                    def _():
                        pltpu.make_async_copy(ob, dst, ws).wait()

                    expand_chunk(cc, ob)
                    pltpu.async_copy(ob, dst, ws)
                return 0

            lax.fori_loop(0, SLAB_CHUNKS // 2, pair_body, 0)
            return 0

        lax.fori_loop(0, slabs, slab_body, 0)
        tail = row0 + rows_per_w - 2 * CHUNK
        for bsel in range(2):
            dst = out_hbm.at[pl.ds(tail + bsel * CHUNK, CHUNK)]
            pltpu.make_async_copy(bufs[bsel], dst, sems[bsel]).wait()

    return k(m_flat, table)


def kernel(m, table):
    b = m.shape[0] * m.shape[1]
    m_flat = m.reshape(b).astype(jnp.int32)
    out = _emb_sc(m_flat, table, b)
    return out.reshape(m.shape[0], m.shape[1], D)
